# baseline (speedup 1.0000x reference)
import functools

import jax
import jax.numpy as jnp
from jax import lax
from jax.experimental import pallas as pl
from jax.experimental.pallas import tpu as pltpu
from jax.experimental.pallas import tpu_sc as plsc

F = 26
V = 1_000_000
VALIGN = 999_936
B = 16384
NC = 2
NS = 16
NW = NC * NS
BPW = B // NW
N = BPW * F
LANES = 16
NCH = BPW // LANES

_mesh = plsc.VectorSubcoreMesh(core_axis_name="c", subcore_axis_name="s")


@functools.partial(
    pl.kernel,
    out_type=jax.ShapeDtypeStruct((B,), jnp.float32),
    mesh=_mesh,
    compiler_params=pltpu.CompilerParams(
        needs_layout_passes=False, use_tc_tiling_on_sc=True
    ),
    scratch_types=[
        pltpu.VMEM((N,), jnp.int32),
        pltpu.VMEM((N,), jnp.float32),
        pltpu.VMEM((BPW,), jnp.float32),
        pltpu.SemaphoreType.DMA,
        pltpu.SemaphoreType.DMA,
    ],
)
def _linear_logits_sc(x_hbm, w_hbm, out_hbm, idxs, vals, accv, sem, xsem):
    wid = lax.axis_index("s") * NC + lax.axis_index("c")
    base = wid * BPW

    idx_copies = [
        pltpu.async_copy(
            x_hbm.at[f, pl.ds(base, BPW)],
            idxs.at[pl.ds(f * BPW, BPW)],
            xsem,
        )
        for f in range(F)
    ]
    copies = []
    for f in range(F):
        seg = pl.ds(f * BPW, BPW)
        idx_copies[f].wait()
        copies.append(
            pltpu.async_copy(
                w_hbm.at[f, 0, pl.ds(0, VALIGN)].at[idxs.at[seg]],
                vals.at[seg],
                sem,
            )
        )
    copies[0].wait()
    copies[1].wait()

    def _init(j, _):
        s = pl.ds(j * LANES, LANES)
        accv[s] = vals[s] + vals[pl.ds(BPW + j * LANES, LANES)]
        return 0

    lax.fori_loop(0, NCH, _init, 0)

    for f in range(2, F):
        copies[f].wait()

        def _acc(j, _, f=f):
            s = pl.ds(j * LANES, LANES)
            accv[s] = accv[s] + vals[pl.ds(f * BPW + j * LANES, LANES)]
            return 0

        lax.fori_loop(0, NCH, _acc, 0)

    pltpu.sync_copy(accv, out_hbm.at[pl.ds(base, BPW)])


def kernel(X, W):
    w_view = jnp.transpose(W, (0, 2, 1))
    x_view = X.T
    out = _linear_logits_sc(x_view, w_view)
    return out.reshape(B, 1)


# speedup vs baseline: 1.0892x; 1.0892x over previous
import functools

import jax
import jax.numpy as jnp
from jax import lax
from jax.experimental import pallas as pl
from jax.experimental.pallas import tpu as pltpu
from jax.experimental.pallas import tpu_sc as plsc

F = 26
V = 1_000_000
VALIGN = 999_936
B = 16384
NC = 2
NS = 16
NW = NC * NS
BPW = B // NW
N = BPW * F
LANES = 16
NCH = BPW // LANES

_mesh = plsc.VectorSubcoreMesh(core_axis_name="c", subcore_axis_name="s")


@functools.partial(
    pl.kernel,
    out_type=jax.ShapeDtypeStruct((B,), jnp.float32),
    mesh=_mesh,
    compiler_params=pltpu.CompilerParams(
        needs_layout_passes=False, use_tc_tiling_on_sc=True
    ),
    scratch_types=[
        pltpu.VMEM((N,), jnp.int32),
        pltpu.VMEM((N,), jnp.float32),
        pltpu.VMEM((BPW,), jnp.float32),
        pltpu.SemaphoreType.DMA,
        pltpu.SemaphoreType.DMA,
    ],
)
def _linear_logits_sc(x_hbm, w_hbm, out_hbm, idxs, vals, accv, sem, xsem):
    wid = lax.axis_index("s") * NC + lax.axis_index("c")
    base = wid * BPW

    idx_copies = [
        pltpu.async_copy(
            x_hbm.at[f, pl.ds(base, BPW)],
            idxs.at[pl.ds(f * BPW, BPW)],
            xsem,
        )
        for f in range(F)
    ]
    copies = []
    for f in range(F):
        seg = pl.ds(f * BPW, BPW)
        idx_copies[f].wait()
        copies.append(
            pltpu.async_copy(
                w_hbm.at[f, 0, pl.ds(0, VALIGN)].at[idxs.at[seg]],
                vals.at[seg],
                sem,
            )
        )
    for c in copies:
        c.wait()

    def _reduce(j, _):
        acc = vals[pl.ds(j * LANES, LANES)]
        for f in range(1, F):
            acc = acc + vals[pl.ds(f * BPW + j * LANES, LANES)]
        accv[pl.ds(j * LANES, LANES)] = acc
        return 0

    lax.fori_loop(0, NCH, _reduce, 0)

    pltpu.sync_copy(accv, out_hbm.at[pl.ds(base, BPW)])


def kernel(X, W):
    w_view = jnp.transpose(W, (0, 2, 1))
    x_view = X.T
    out = _linear_logits_sc(x_view, w_view)
    return out.reshape(B, 1)


# speedup vs baseline: 1.0896x; 1.0003x over previous
import functools

import jax
import jax.numpy as jnp
from jax import lax
from jax.experimental import pallas as pl
from jax.experimental.pallas import tpu as pltpu
from jax.experimental.pallas import tpu_sc as plsc

F = 26
V = 1_000_000
VALIGN = 999_936
B = 16384
NC = 2
NS = 16
NW = NC * NS
BPW = B // NW
N = BPW * F
LANES = 16
NCH = BPW // LANES

_mesh = plsc.VectorSubcoreMesh(core_axis_name="c", subcore_axis_name="s")


@functools.partial(
    pl.kernel,
    out_type=jax.ShapeDtypeStruct((B,), jnp.float32),
    mesh=_mesh,
    compiler_params=pltpu.CompilerParams(
        needs_layout_passes=False, use_tc_tiling_on_sc=True
    ),
    scratch_types=[
        pltpu.VMEM((N,), jnp.int32),
        pltpu.VMEM((N,), jnp.float32),
        pltpu.VMEM((BPW,), jnp.float32),
        pltpu.SemaphoreType.DMA,
        pltpu.SemaphoreType.DMA,
    ],
)
def _linear_logits_sc(x_hbm, w_hbm, out_hbm, idxs, vals, accv, sem, xsem):
    wid = lax.axis_index("s") * NC + lax.axis_index("c")
    base = wid * BPW

    idx_copies = [
        pltpu.async_copy(
            x_hbm.at[f, pl.ds(base, BPW)],
            idxs.at[pl.ds(f * BPW, BPW)],
            xsem,
        )
        for f in range(F)
    ]
    copies = []
    for f in range(F):
        seg = pl.ds(f * BPW, BPW)
        idx_copies[f].wait()
        copies.append(
            pltpu.async_copy(
                w_hbm.at[f, 0, pl.ds(0, VALIGN)].at[idxs.at[seg]],
                vals.at[seg],
                sem,
            )
        )
    HALF = 13
    for c in copies[:HALF]:
        c.wait()

    def _reduce_lo(j, _):
        acc = vals[pl.ds(j * LANES, LANES)]
        for f in range(1, HALF):
            acc = acc + vals[pl.ds(f * BPW + j * LANES, LANES)]
        accv[pl.ds(j * LANES, LANES)] = acc
        return 0

    lax.fori_loop(0, NCH, _reduce_lo, 0)

    for c in copies[HALF:]:
        c.wait()

    def _reduce_hi(j, _):
        acc = accv[pl.ds(j * LANES, LANES)]
        for f in range(HALF, F):
            acc = acc + vals[pl.ds(f * BPW + j * LANES, LANES)]
        accv[pl.ds(j * LANES, LANES)] = acc
        return 0

    lax.fori_loop(0, NCH, _reduce_hi, 0)

    pltpu.sync_copy(accv, out_hbm.at[pl.ds(base, BPW)])


def kernel(X, W):
    w_view = jnp.transpose(W, (0, 2, 1))
    x_view = X.T
    out = _linear_logits_sc(x_view, w_view)
    return out.reshape(B, 1)
